# Initial kernel scaffold; baseline (speedup 1.0000x reference)
import jax
import jax.numpy as jnp
from jax.experimental import pallas as pl


def kernel(x, edge_index, edge_attr, d2an, Wnode1, W1_conv1, W1_conv3):
    raise NotImplementedError("write your pallas kernel here")


# speedup vs baseline: 18.3646x; 18.3646x over previous
import functools

import jax
import jax.numpy as jnp
from jax import lax
from jax.experimental import pallas as pl
from jax.experimental.pallas import tpu as pltpu
from jax.experimental.pallas import tpu_sc as plsc

N = 10000
E = 320000
D = 128

NC = 2
NS = 16
NW = NC * NS

C = 125
ECHUNKS = E // C
G = ECHUNKS // NW
WB = N // 10

_MESH = plsc.VectorSubcoreMesh(
    core_axis_name="c", subcore_axis_name="s", num_cores=NC, num_subcores=NS
)


@functools.partial(
    pl.kernel,
    mesh=_MESH,
    out_type=jax.ShapeDtypeStruct((NC, N, D), jnp.float32),
    scratch_types=[
        pltpu.VMEM((G, C), jnp.int32),
        pltpu.VMEM((C, D), jnp.float32),
        pltpu.VMEM_SHARED((N, D), jnp.float32),
    ],
)
def _sc_degree(col_hbm, ones_hbm, zn_hbm, out_hbm, col_v, ones_v, acc):
    c = lax.axis_index("c")
    s = lax.axis_index("s")
    w = c * NS + s

    @pl.when(s < 10)
    def _zero():
        pltpu.sync_copy(zn_hbm.at[pl.ds(s * WB, WB)], acc.at[pl.ds(s * WB, WB)])

    pltpu.sync_copy(ones_hbm, ones_v)
    pltpu.sync_copy(col_hbm.at[pl.ds(w * G, G)], col_v)
    plsc.subcore_barrier()

    def body(g, carry):
        pltpu.sync_copy(ones_v, acc.at[col_v.at[g]], add=True)
        return carry

    lax.fori_loop(0, G, body, 0)
    plsc.subcore_barrier()

    @pl.when(s < 10)
    def _wb():
        pltpu.sync_copy(acc.at[pl.ds(s * WB, WB)], out_hbm.at[c].at[pl.ds(s * WB, WB)])


@functools.partial(
    pl.kernel,
    mesh=_MESH,
    out_type=jax.ShapeDtypeStruct((NC, N, D), jnp.float32),
    scratch_types=[
        pltpu.VMEM((G, C), jnp.int32),
        pltpu.VMEM((G, C), jnp.int32),
        pltpu.VMEM((C, D), jnp.float32),
        pltpu.VMEM_SHARED((N, D), jnp.float32),
        pltpu.SemaphoreType.DMA,
    ],
)
def _sc_scatter(ys_hbm, row_hbm, col_hbm, znd_hbm, out_hbm, row_v, col_v, rows_v, acc, sem):
    c = lax.axis_index("c")
    s = lax.axis_index("s")
    w = c * NS + s

    @pl.when(s < 10)
    def _zero():
        pltpu.sync_copy(znd_hbm.at[pl.ds(s * WB, WB)], acc.at[pl.ds(s * WB, WB)])

    pltpu.sync_copy(row_hbm.at[pl.ds(w * G, G)], row_v)
    pltpu.sync_copy(col_hbm.at[pl.ds(w * G, G)], col_v)
    plsc.subcore_barrier()

    def body(g, carry):
        pltpu.async_copy(ys_hbm.at[row_v.at[g]], rows_v, sem).wait()
        pltpu.sync_copy(rows_v, acc.at[col_v.at[g]], add=True)
        return carry

    lax.fori_loop(0, G, body, 0)
    plsc.subcore_barrier()

    @pl.when(s < 10)
    def _wb():
        pltpu.sync_copy(acc.at[pl.ds(s * WB, WB)], out_hbm.at[c].at[pl.ds(s * WB, WB)])


_R = 2000


def _dis_body(dp_ref, o_ref):
    deg = 1.0 + dp_ref[0][:, 0:1] + dp_ref[1][:, 0:1]
    o_ref[...] = lax.rsqrt(deg)


def _tc_dis(degp):
    return pl.pallas_call(
        _dis_body,
        grid=(N // _R,),
        in_specs=[pl.BlockSpec((NC, _R, D), lambda i: (0, i, 0))],
        out_specs=pl.BlockSpec((_R, 1), lambda i: (i, 0)),
        out_shape=jax.ShapeDtypeStruct((N, 1), jnp.float32),
    )(degp)


def _mm1_body(x_ref, d_ref, wa_ref, wb_ref, wc_ref, dis_ref, o_ref):
    t = jnp.dot(x_ref[...], wa_ref[...], preferred_element_type=jnp.float32)
    t += jnp.dot(d_ref[...], wb_ref[...], preferred_element_type=jnp.float32)
    y = jnp.dot(t, wc_ref[...], preferred_element_type=jnp.float32)
    o_ref[...] = dis_ref[...] * y


def _tc_layer1(x, d2an_p, wa, wb_p, wc1, dis):
    return pl.pallas_call(
        _mm1_body,
        grid=(N // _R,),
        in_specs=[
            pl.BlockSpec((_R, D), lambda i: (i, 0)),
            pl.BlockSpec((_R, D), lambda i: (i, 0)),
            pl.BlockSpec((D, D), lambda i: (0, 0)),
            pl.BlockSpec((D, D), lambda i: (0, 0)),
            pl.BlockSpec((D, D), lambda i: (0, 0)),
            pl.BlockSpec((_R, 1), lambda i: (i, 0)),
        ],
        out_specs=pl.BlockSpec((_R, D), lambda i: (i, 0)),
        out_shape=jax.ShapeDtypeStruct((N, D), jnp.float32),
    )(x, d2an_p, wa, wb_p, wc1, dis)


def _mid_body(ap_ref, ys_ref, dis_ref, wc_ref, o_ref):
    dis = dis_ref[...]
    h = jnp.maximum(dis * (ap_ref[0] + ap_ref[1] + ys_ref[...]), 0.0)
    o_ref[...] = dis * jnp.dot(h, wc_ref[...], preferred_element_type=jnp.float32)


def _tc_mid(accp, ys1, dis, wc3):
    return pl.pallas_call(
        _mid_body,
        grid=(N // _R,),
        in_specs=[
            pl.BlockSpec((NC, _R, D), lambda i: (0, i, 0)),
            pl.BlockSpec((_R, D), lambda i: (i, 0)),
            pl.BlockSpec((_R, 1), lambda i: (i, 0)),
            pl.BlockSpec((D, D), lambda i: (0, 0)),
        ],
        out_specs=pl.BlockSpec((_R, D), lambda i: (i, 0)),
        out_shape=jax.ShapeDtypeStruct((N, D), jnp.float32),
    )(accp, ys1, dis, wc3)


def _final_body(ap_ref, ys_ref, dis_ref, o_ref):
    o_ref[...] = jnp.maximum(
        dis_ref[...] * (ap_ref[0] + ap_ref[1] + ys_ref[...]), 0.0
    )


def _tc_final(accp, ys2, dis):
    return pl.pallas_call(
        _final_body,
        grid=(N // _R,),
        in_specs=[
            pl.BlockSpec((NC, _R, D), lambda i: (0, i, 0)),
            pl.BlockSpec((_R, D), lambda i: (i, 0)),
            pl.BlockSpec((_R, 1), lambda i: (i, 0)),
        ],
        out_specs=pl.BlockSpec((_R, D), lambda i: (i, 0)),
        out_shape=jax.ShapeDtypeStruct((N, D), jnp.float32),
    )(accp, ys2, dis)


def kernel(x, edge_index, edge_attr, d2an, Wnode1, W1_conv1, W1_conv3):
    del edge_attr
    ei = edge_index.astype(jnp.int32)
    row2d = ei[0].reshape(ECHUNKS, C)
    col2d = ei[1].reshape(ECHUNKS, C)

    d2an_p = jnp.pad(d2an, ((0, 0), (0, D - d2an.shape[1])))
    wa = Wnode1[:D]
    wb_p = jnp.pad(Wnode1[D:], ((0, D - (Wnode1.shape[0] - D)), (0, 0)))

    znd = jnp.zeros((N, D), jnp.float32)
    ones_c = jnp.ones((C, D), jnp.float32)

    degp = _sc_degree(col2d, ones_c, znd)
    dis = _tc_dis(degp)
    ys1 = _tc_layer1(x, d2an_p, wa, wb_p, W1_conv1, dis)
    accp1 = _sc_scatter(ys1, row2d, col2d, znd)
    ys2 = _tc_mid(accp1, ys1, dis, W1_conv3)
    accp2 = _sc_scatter(ys2, row2d, col2d, znd)
    return _tc_final(accp2, ys2, dis)


# speedup vs baseline: 25.1251x; 1.3681x over previous
import functools

import jax
import jax.numpy as jnp
from jax import lax
from jax.experimental import pallas as pl
from jax.experimental.pallas import tpu as pltpu
from jax.experimental.pallas import tpu_sc as plsc

N = 10000
E = 320000
D = 128

NC = 2
NS = 16
NW = NC * NS

C = 125
ECHUNKS = E // C
G = ECHUNKS // NW
WB = N // 10

_MESH = plsc.VectorSubcoreMesh(
    core_axis_name="c", subcore_axis_name="s", num_cores=NC, num_subcores=NS
)


@functools.partial(
    pl.kernel,
    mesh=_MESH,
    out_type=jax.ShapeDtypeStruct((NC, N, D), jnp.float32),
    scratch_types=[
        pltpu.VMEM((G, C), jnp.int32),
        pltpu.VMEM((C, D), jnp.float32),
        pltpu.VMEM_SHARED((N, D), jnp.float32),
        pltpu.SemaphoreType.DMA,
    ],
)
def _sc_degree(col_hbm, ones_hbm, zn_hbm, out_hbm, col_v, ones_v, acc, sem):
    c = lax.axis_index("c")
    s = lax.axis_index("s")
    w = c * NS + s

    @pl.when(s < 10)
    def _zero():
        pltpu.sync_copy(zn_hbm.at[pl.ds(s * WB, WB)], acc.at[pl.ds(s * WB, WB)])

    pltpu.sync_copy(ones_hbm, ones_v)
    pltpu.sync_copy(col_hbm.at[pl.ds(w * G, G)], col_v)
    plsc.subcore_barrier()

    LAG = 8

    def fire(g, carry):
        pltpu.async_copy(ones_v, acc.at[col_v.at[g]], sem, add=True)

        @pl.when(g >= LAG)
        def _():
            pltpu.make_async_copy(ones_v, acc.at[col_v.at[g - LAG]], sem).wait()

        return carry

    lax.fori_loop(0, G, fire, 0)

    def drain(g, carry):
        pltpu.make_async_copy(ones_v, acc.at[col_v.at[g]], sem).wait()
        return carry

    lax.fori_loop(G - LAG, G, drain, 0)
    plsc.subcore_barrier()

    @pl.when(s < 10)
    def _wb():
        pltpu.sync_copy(acc.at[pl.ds(s * WB, WB)], out_hbm.at[c].at[pl.ds(s * WB, WB)])


@functools.partial(
    pl.kernel,
    mesh=_MESH,
    out_type=jax.ShapeDtypeStruct((NC, N, D), jnp.float32),
    scratch_types=[
        pltpu.VMEM((G // 2, C), jnp.int32),
        pltpu.VMEM((G // 2, C), jnp.int32),
        pltpu.VMEM((C, D), jnp.float32),
        pltpu.VMEM((C, D), jnp.float32),
        pltpu.VMEM_SHARED((N, D), jnp.float32),
        pltpu.SemaphoreType.DMA,
        pltpu.SemaphoreType.DMA,
    ],
)
def _sc_scatter(ys_hbm, row_hbm, col_hbm, znd_hbm, out_hbm,
                row_v, col_v, rows_v0, rows_v1, acc, sem0, sem1):
    c = lax.axis_index("c")
    s = lax.axis_index("s")
    w = c * NS + s

    @pl.when(s < 10)
    def _zero():
        pltpu.sync_copy(znd_hbm.at[pl.ds(s * WB, WB)], acc.at[pl.ds(s * WB, WB)])

    plsc.subcore_barrier()

    H = G // 2
    for h in range(2):
        pltpu.sync_copy(row_hbm.at[pl.ds(w * G + h * H, H)], row_v)
        pltpu.sync_copy(col_hbm.at[pl.ds(w * G + h * H, H)], col_v)
        pltpu.async_copy(ys_hbm.at[row_v.at[0]], rows_v0, sem0)

        def body(i, carry):
            g = 2 * i
            pltpu.async_copy(ys_hbm.at[row_v.at[g + 1]], rows_v1, sem1)
            pltpu.make_async_copy(ys_hbm.at[row_v.at[g]], rows_v0, sem0).wait()
            pltpu.sync_copy(rows_v0, acc.at[col_v.at[g]], add=True)

            @pl.when(g + 2 < H)
            def _pf():
                pltpu.async_copy(ys_hbm.at[row_v.at[g + 2]], rows_v0, sem0)

            pltpu.make_async_copy(ys_hbm.at[row_v.at[g + 1]], rows_v1, sem1).wait()
            pltpu.sync_copy(rows_v1, acc.at[col_v.at[g + 1]], add=True)
            return carry

        lax.fori_loop(0, H // 2, body, 0)
    plsc.subcore_barrier()

    @pl.when(s < 10)
    def _wb():
        pltpu.sync_copy(acc.at[pl.ds(s * WB, WB)], out_hbm.at[c].at[pl.ds(s * WB, WB)])


_R = 2000


def _dis_body(dp_ref, o_ref):
    deg = 1.0 + dp_ref[0][:, 0:1] + dp_ref[1][:, 0:1]
    o_ref[...] = lax.rsqrt(deg)


def _tc_dis(degp):
    return pl.pallas_call(
        _dis_body,
        grid=(N // _R,),
        in_specs=[pl.BlockSpec((NC, _R, D), lambda i: (0, i, 0))],
        out_specs=pl.BlockSpec((_R, 1), lambda i: (i, 0)),
        out_shape=jax.ShapeDtypeStruct((N, 1), jnp.float32),
    )(degp)


def _mm1_body(x_ref, d_ref, wa_ref, wb_ref, wc_ref, dis_ref, o_ref):
    t = jnp.dot(x_ref[...], wa_ref[...], preferred_element_type=jnp.float32)
    t += jnp.dot(d_ref[...], wb_ref[...], preferred_element_type=jnp.float32)
    y = jnp.dot(t, wc_ref[...], preferred_element_type=jnp.float32)
    o_ref[...] = dis_ref[...] * y


def _tc_layer1(x, d2an_p, wa, wb_p, wc1, dis):
    return pl.pallas_call(
        _mm1_body,
        grid=(N // _R,),
        in_specs=[
            pl.BlockSpec((_R, D), lambda i: (i, 0)),
            pl.BlockSpec((_R, D), lambda i: (i, 0)),
            pl.BlockSpec((D, D), lambda i: (0, 0)),
            pl.BlockSpec((D, D), lambda i: (0, 0)),
            pl.BlockSpec((D, D), lambda i: (0, 0)),
            pl.BlockSpec((_R, 1), lambda i: (i, 0)),
        ],
        out_specs=pl.BlockSpec((_R, D), lambda i: (i, 0)),
        out_shape=jax.ShapeDtypeStruct((N, D), jnp.float32),
    )(x, d2an_p, wa, wb_p, wc1, dis)


def _mid_body(ap_ref, ys_ref, dis_ref, wc_ref, o_ref):
    dis = dis_ref[...]
    h = jnp.maximum(dis * (ap_ref[0] + ap_ref[1] + ys_ref[...]), 0.0)
    o_ref[...] = dis * jnp.dot(h, wc_ref[...], preferred_element_type=jnp.float32)


def _tc_mid(accp, ys1, dis, wc3):
    return pl.pallas_call(
        _mid_body,
        grid=(N // _R,),
        in_specs=[
            pl.BlockSpec((NC, _R, D), lambda i: (0, i, 0)),
            pl.BlockSpec((_R, D), lambda i: (i, 0)),
            pl.BlockSpec((_R, 1), lambda i: (i, 0)),
            pl.BlockSpec((D, D), lambda i: (0, 0)),
        ],
        out_specs=pl.BlockSpec((_R, D), lambda i: (i, 0)),
        out_shape=jax.ShapeDtypeStruct((N, D), jnp.float32),
    )(accp, ys1, dis, wc3)


def _final_body(ap_ref, ys_ref, dis_ref, o_ref):
    o_ref[...] = jnp.maximum(
        dis_ref[...] * (ap_ref[0] + ap_ref[1] + ys_ref[...]), 0.0
    )


def _tc_final(accp, ys2, dis):
    return pl.pallas_call(
        _final_body,
        grid=(N // _R,),
        in_specs=[
            pl.BlockSpec((NC, _R, D), lambda i: (0, i, 0)),
            pl.BlockSpec((_R, D), lambda i: (i, 0)),
            pl.BlockSpec((_R, 1), lambda i: (i, 0)),
        ],
        out_specs=pl.BlockSpec((_R, D), lambda i: (i, 0)),
        out_shape=jax.ShapeDtypeStruct((N, D), jnp.float32),
    )(accp, ys2, dis)


def kernel(x, edge_index, edge_attr, d2an, Wnode1, W1_conv1, W1_conv3):
    del edge_attr
    ei = edge_index.astype(jnp.int32)
    row2d = ei[0].reshape(ECHUNKS, C)
    col2d = ei[1].reshape(ECHUNKS, C)

    d2an_p = jnp.pad(d2an, ((0, 0), (0, D - d2an.shape[1])))
    wa = Wnode1[:D]
    wb_p = jnp.pad(Wnode1[D:], ((0, D - (Wnode1.shape[0] - D)), (0, 0)))

    znd = jnp.zeros((N, D), jnp.float32)
    ones_c = jnp.ones((C, D), jnp.float32)

    degp = _sc_degree(col2d, ones_c, znd)
    dis = _tc_dis(degp)
    ys1 = _tc_layer1(x, d2an_p, wa, wb_p, W1_conv1, dis)
    accp1 = _sc_scatter(ys1, row2d, col2d, znd)
    ys2 = _tc_mid(accp1, ys1, dis, W1_conv3)
    accp2 = _sc_scatter(ys2, row2d, col2d, znd)
    return _tc_final(accp2, ys2, dis)
